# baseline (speedup 1.0000x reference)
import functools

import jax
import jax.numpy as jnp
from jax import lax
from jax.experimental import pallas as pl
from jax.experimental.pallas import tpu as pltpu
from jax.experimental.pallas import tpu_sc as plsc

LANES = 16
NBUF = 2


def _tc_detile(wt, blk_i=512):
    emb_dim, n_emb = wt.shape

    def body(in_ref, out_ref):
        t = in_ref[...].T
        out_ref[...] = jnp.concatenate([t, t], axis=1)

    return pl.pallas_call(
        body,
        grid=(pl.cdiv(n_emb, blk_i),),
        in_specs=[pl.BlockSpec((emb_dim, blk_i), lambda i: (0, i))],
        out_specs=pl.BlockSpec((blk_i, 2 * emb_dim), lambda i: (i, 0)),
        out_shape=jax.ShapeDtypeStruct((n_emb, 2 * emb_dim), jnp.float32),
    )(wt)


def _build_gather(bsz, seq, emb_dim, n_workers):
    bt = bsz // n_workers
    rpc = 2 * bt
    assert bt % 8 == 0 and bt <= 128

    mesh = plsc.VectorSubcoreMesh(core_axis_name="c", subcore_axis_name="s")

    @functools.partial(
        pl.kernel,
        mesh=mesh,
        compiler_params=pltpu.CompilerParams(
            needs_layout_passes=False, use_tc_tiling_on_sc=False),
        out_type=jax.ShapeDtypeStruct((seq, emb_dim, n_workers, rpc),
                                      jnp.float32),
        scratch_types=[
            pltpu.VMEM((seq, rpc), jnp.int32),
            pltpu.VMEM((NBUF, rpc, emb_dim), jnp.float32),
            pltpu.VMEM((NBUF, emb_dim, rpc + 1), jnp.float32),
            pltpu.SemaphoreType.DMA,
            pltpu.SemaphoreType.DMA,
            pltpu.SemaphoreType.DMA,
            pltpu.SemaphoreType.DMA,
        ],
    )
    def gather_kernel(idx_hbm, w_hbm, out_hbm, idx_v, rows_v, out_v,
                      gsem0, gsem1, osem0, osem1):
        n_cores = mesh.num_cores
        wid = lax.axis_index("s") * n_cores + lax.axis_index("c")

        gsems = (gsem0, gsem1)
        osems = (osem0, osem1)

        lane = lax.iota(jnp.int32, 16)

        pltpu.sync_copy(idx_hbm.at[:, wid], idx_v)

        def dbl_body(i, carry):
            for g in range(rpc // LANES):
                sl = pl.ds(g * LANES, LANES)
                idx_v[i, sl] = 2 * idx_v[i, sl]
            return carry

        lax.fori_loop(0, seq, dbl_body, 0, unroll=4)

        def start_gather(c, b):
            for h in range(2):
                pltpu.async_copy(
                    w_hbm.at[idx_v.at[c, pl.ds(h * bt, bt)]],
                    rows_v.at[b, pl.ds(h * bt, bt)], gsems[b])

        def wait_gather(c, b):
            for h in range(2):
                pltpu.make_async_copy(
                    w_hbm.at[idx_v.at[c, pl.ds(h * bt, bt)]],
                    rows_v.at[b, pl.ds(h * bt, bt)], gsems[b]).wait()

        def start_out(c, b):
            pltpu.async_copy(out_v.at[b, :, pl.ds(0, rpc)],
                             out_hbm.at[c, :, wid], osems[b])

        def wait_out(c, b):
            pltpu.make_async_copy(
                out_v.at[b, :, pl.ds(0, rpc)],
                out_hbm.at[c, :, wid], osems[b]).wait()

        for b in range(NBUF):
            start_gather(b, b)

        def outer(g, carry):
            for b in range(NBUF):
                c = g * NBUF + b
                wait_gather(c, b)

                @pl.when(g > 0)
                def _():
                    wait_out(c - NBUF, b)

                def row_body(r, carry2):
                    vr = jnp.full((LANES,), r, jnp.int32)
                    vals = [rows_v[b, r, pl.ds(grp * LANES, LANES)]
                            for grp in range(emb_dim // LANES)]
                    for grp, a in enumerate(vals):
                        plsc.store_scatter(
                            out_v.at[b], [grp * LANES + lane, vr], a)
                    return carry2

                lax.fori_loop(0, rpc, row_body, 0, unroll=8)
                start_out(c, b)

                @pl.when(c + NBUF < seq)
                def _():
                    start_gather(c + NBUF, b)
            return carry

        lax.fori_loop(0, seq // NBUF, outer, 0)

        for b in range(NBUF):
            wait_out(seq - NBUF + b, b)

    return gather_kernel


def kernel(avnn_tensor, weight):
    bsz, seq, two = avnn_tensor.shape
    assert two == 2
    emb_dim = weight.shape[1]
    n_workers = 32
    bt = bsz // n_workers

    idx = avnn_tensor.astype(jnp.int32)
    idx = idx.transpose(1, 0, 2).reshape(seq, n_workers, bt, 2)
    idx = idx.transpose(0, 1, 3, 2).reshape(seq, n_workers, 2 * bt)

    table = _tc_detile(weight.T).reshape(2 * weight.shape[0], emb_dim)

    gather = _build_gather(bsz, seq, emb_dim, n_workers)
    out4 = gather(idx, table)

    out = out4.reshape(seq, emb_dim, n_workers, 2, bt)
    out = out.transpose(2, 4, 0, 1, 3).reshape(bsz, seq, emb_dim, 2)
    return out


# speedup vs baseline: 1.5686x; 1.5686x over previous
import functools

import jax
import jax.numpy as jnp
from jax import lax
from jax.experimental import pallas as pl
from jax.experimental.pallas import tpu as pltpu
from jax.experimental.pallas import tpu_sc as plsc

LANES = 16
NBUF = 2


def _build_gather(bsz, seq, emb_dim, n_workers):
    bt = bsz // n_workers
    rpc = 2 * bt
    assert bt % 8 == 0 and bt <= 128

    mesh = plsc.VectorSubcoreMesh(core_axis_name="c", subcore_axis_name="s")

    @functools.partial(
        pl.kernel,
        mesh=mesh,
        compiler_params=pltpu.CompilerParams(
            needs_layout_passes=False, use_tc_tiling_on_sc=False),
        out_type=jax.ShapeDtypeStruct((seq, emb_dim, n_workers, rpc),
                                      jnp.float32),
        scratch_types=[
            pltpu.VMEM((seq, rpc), jnp.int32),
            pltpu.VMEM((NBUF, rpc, emb_dim), jnp.float32),
            pltpu.VMEM((NBUF, emb_dim, rpc + 1), jnp.float32),
            pltpu.SemaphoreType.DMA,
            pltpu.SemaphoreType.DMA,
            pltpu.SemaphoreType.DMA,
            pltpu.SemaphoreType.DMA,
        ],
    )
    def gather_kernel(idx_hbm, w_hbm, out_hbm, idx_v, rows_v, out_v,
                      gsem0, gsem1, osem0, osem1):
        n_cores = mesh.num_cores
        wid = lax.axis_index("s") * n_cores + lax.axis_index("c")

        gsems = (gsem0, gsem1)
        osems = (osem0, osem1)

        lane = lax.iota(jnp.int32, 16)

        pltpu.sync_copy(idx_hbm.at[:, wid], idx_v)

        def start_gather(c, b):
            for h in range(2):
                pltpu.async_copy(
                    w_hbm.at[idx_v.at[c, pl.ds(h * bt, bt)]],
                    rows_v.at[b, pl.ds(h * bt, bt)], gsems[b])

        def wait_gather(c, b):
            for h in range(2):
                pltpu.make_async_copy(
                    w_hbm.at[idx_v.at[c, pl.ds(h * bt, bt)]],
                    rows_v.at[b, pl.ds(h * bt, bt)], gsems[b]).wait()

        def start_out(c, b):
            pltpu.async_copy(out_v.at[b, :, pl.ds(0, rpc)],
                             out_hbm.at[c, :, wid], osems[b])

        def wait_out(c, b):
            pltpu.make_async_copy(
                out_v.at[b, :, pl.ds(0, rpc)],
                out_hbm.at[c, :, wid], osems[b]).wait()

        for b in range(NBUF):
            start_gather(b, b)

        def outer(g, carry):
            for b in range(NBUF):
                c = g * NBUF + b
                wait_gather(c, b)

                @pl.when(g > 0)
                def _():
                    wait_out(c - NBUF, b)

                def row_body(r, carry2):
                    vr = jnp.full((LANES,), r, jnp.int32)
                    vals = [rows_v[b, r, pl.ds(grp * LANES, LANES)]
                            for grp in range(emb_dim // LANES)]
                    for grp, a in enumerate(vals):
                        plsc.store_scatter(
                            out_v.at[b], [grp * LANES + lane, vr], a)
                    return carry2

                lax.fori_loop(0, rpc, row_body, 0, unroll=8)
                start_out(c, b)

                @pl.when(c + NBUF < seq)
                def _():
                    start_gather(c + NBUF, b)
            return carry

        lax.fori_loop(0, seq // NBUF, outer, 0)

        for b in range(NBUF):
            wait_out(seq - NBUF + b, b)

    return gather_kernel


def kernel(avnn_tensor, weight):
    bsz, seq, two = avnn_tensor.shape
    assert two == 2
    emb_dim = weight.shape[1]
    n_workers = 32
    bt = bsz // n_workers

    idx = avnn_tensor.astype(jnp.int32)
    idx = idx.transpose(1, 0, 2).reshape(seq, n_workers, bt, 2)
    idx = idx.transpose(0, 1, 3, 2).reshape(seq, n_workers, 2 * bt)

    gather = _build_gather(bsz, seq, emb_dim, n_workers)
    out4 = gather(idx, weight)

    out = out4.reshape(seq, emb_dim, n_workers, 2, bt)
    out = out.transpose(2, 4, 0, 1, 3).reshape(bsz, seq, emb_dim, 2)
    return out
